# Initial kernel scaffold; baseline (speedup 1.0000x reference)
import jax
import jax.numpy as jnp
from jax.experimental import pallas as pl


def kernel(pred, target, sampled_coords):
    raise NotImplementedError("write your pallas kernel here")


# speedup vs baseline: 1.0660x; 1.0660x over previous
import functools

import jax
import jax.numpy as jnp
from jax import lax
from jax.experimental import pallas as pl
from jax.experimental.pallas import tpu as pltpu
from jax.experimental.pallas import tpu_sc as plsc

_B = 16
_H = 512
_W = 512
_S = 4096
_NW = 32
_SPW = (_B * _S) // _NW
_LANES = 16
_CHUNKS = _SPW // _LANES
_GCH = 128
_NGATHER = _SPW // _GCH
_ROWS_PER_IMG = (_H * _W) // _LANES


def _round_half_even_idx(x_f32, limit):
    f = x_f32.astype(jnp.int32)
    ff = f.astype(jnp.float32)
    d = x_f32 - ff
    half = jnp.float32(0.5)
    odd = (f & 1) == 1
    up = (d > half) | ((d == half) & odd)
    r = f + jnp.where(up, jnp.int32(1), jnp.int32(0))
    return jnp.clip(r, 0, limit)


def _sc_body(pred_hbm, tgt_hbm, u_hbm, v_hbm, out_hbm,
             u_v, v_v, row_v, lane_v, prow_v, trow_v, out_v, sem):
    wid = lax.axis_index("s") * 2 + lax.axis_index("c")
    b = wid // 2
    base = wid * _SPW

    pltpu.sync_copy(u_hbm.at[pl.ds(base, _SPW)], u_v)
    pltpu.sync_copy(v_hbm.at[pl.ds(base, _SPW)], v_v)

    row_base = b * _ROWS_PER_IMG

    def idx_body(i, _):
        off = i * _LANES
        u16 = u_v[pl.ds(off, _LANES)]
        v16 = v_v[pl.ds(off, _LANES)]
        x = _round_half_even_idx(u16 * jnp.float32(_W - 1), _W - 1)
        y = _round_half_even_idx(v16 * jnp.float32(_H - 1), _H - 1)
        flat = y * _W + x
        row_v[pl.ds(off, _LANES)] = row_base + (flat >> 4)
        lane_v[pl.ds(off, _LANES)] = flat & 15
        return _

    lax.fori_loop(0, _CHUNKS, idx_body, None)

    copies = []
    for j in range(_NGATHER):
        sl = pl.ds(j * _GCH, _GCH)
        copies.append(pltpu.async_copy(pred_hbm.at[row_v.at[sl]],
                                       prow_v.at[sl], sem))
        copies.append(pltpu.async_copy(tgt_hbm.at[row_v.at[sl]],
                                       trow_v.at[sl], sem))
    for c in copies:
        c.wait()

    def acc_body(i, acc):
        off = i * _LANES
        idx0 = lax.iota(jnp.int32, _LANES) + off
        lanes = lane_v[pl.ds(off, _LANES)]
        pv = plsc.load_gather(prow_v, [idx0, lanes])
        tv = plsc.load_gather(trow_v, [idx0, lanes])
        d = pv - tv
        return acc + d * d

    acc = lax.fori_loop(0, _CHUNKS, acc_body,
                        jnp.zeros((_LANES,), jnp.float32))
    total = jnp.sum(acc, axis=0)
    out_v[...] = jnp.full((_LANES,), total, jnp.float32)
    pltpu.sync_copy(out_v, out_hbm.at[wid])


_sc_gather_mse = functools.partial(
    pl.kernel,
    mesh=plsc.VectorSubcoreMesh(core_axis_name="c", subcore_axis_name="s"),
    out_type=jax.ShapeDtypeStruct((_NW, _LANES), jnp.float32),
    scratch_types=[
        pltpu.VMEM((_SPW,), jnp.float32),
        pltpu.VMEM((_SPW,), jnp.float32),
        pltpu.VMEM((_SPW,), jnp.int32),
        pltpu.VMEM((_SPW,), jnp.int32),
        pltpu.VMEM((_SPW, _LANES), jnp.float32),
        pltpu.VMEM((_SPW, _LANES), jnp.float32),
        pltpu.VMEM((_LANES,), jnp.float32),
        pltpu.SemaphoreType.DMA,
    ],
    compiler_params=pltpu.CompilerParams(needs_layout_passes=False,
                                         use_tc_tiling_on_sc=False),
)(_sc_body)


def _finish_body(p_ref, o_ref):
    o_ref[0, 0] = jnp.sum(p_ref[:, 0:1]) * jnp.float32(1.0 / (_B * _S))


_finish = pl.pallas_call(
    _finish_body,
    out_shape=jax.ShapeDtypeStruct((1, 1), jnp.float32),
    out_specs=pl.BlockSpec(memory_space=pltpu.SMEM),
)


def kernel(pred, target, sampled_coords):
    pred2d = pred.reshape(_B * _ROWS_PER_IMG, _LANES)
    tgt2d = target.reshape(_B * _ROWS_PER_IMG, _LANES)
    u = sampled_coords[:, :, 0].reshape(_B * _S)
    v = sampled_coords[:, :, 1].reshape(_B * _S)
    partials = _sc_gather_mse(pred2d, tgt2d, u, v)
    return _finish(partials)[0, 0]
